# baseline (speedup 1.0000x reference)
import functools

import jax
import jax.numpy as jnp
from jax import lax
from jax.experimental import pallas as pl
from jax.experimental.pallas import tpu as pltpu
from jax.experimental.pallas import tpu_sc as plsc

NC = 2
NS = 16
L = 16
NW = NC * NS

D = 50
BLK = 64


def _sc_body(b_per_w, tidx_hbm, utab_hbm, mtab_hbm, w4_hbm, b_hbm,
             out_hbm, idx2_v, uid_v, mid_v, dat_ua, dat_ma, dat_ub, dat_mb,
             out_v, mat_v, w4_v, b_v, sem_u, sem_m, sem_u2, sem_m2):
    wid = lax.axis_index("s") * NC + lax.axis_index("c")
    base = wid * b_per_w

    pltpu.sync_copy(tidx_hbm.at[pl.ds(2 * base, 2 * b_per_w)], idx2_v)
    pltpu.sync_copy(w4_hbm, w4_v)
    pltpu.sync_copy(b_hbm, b_v)

    lanes2 = lax.iota(jnp.int32, L) * 2
    for j in range(b_per_w // L):
        off = j * 2 * L
        uid_v[pl.ds(j * L, L)] = plsc.load_gather(idx2_v, [lanes2 + off])
        mid_v[pl.ds(j * L, L)] = plsc.load_gather(idx2_v,
                                                  [lanes2 + (off + 1)])

    wa = w4_v[pl.ds(0, L)]
    wb = w4_v[pl.ds(L, L)]
    wc = w4_v[pl.ds(2 * L, L)]
    wd = w4_v[pl.ds(3 * L, L)]
    bias = b_v[...]
    base16 = lax.iota(jnp.int32, L) * L
    c0 = lax.iota(jnp.int32, L)
    c1 = c0 + L
    c2 = c0 + 2 * L
    c3 = c0 + (D - L)

    def fire_block(b0, du, dm, su, sm):
        def fire(p, _):
            pv = jnp.full((L,), b0 + p, jnp.int32)
            uid = jnp.max(plsc.load_gather(uid_v, [pv]))
            mid = jnp.max(plsc.load_gather(mid_v, [pv]))
            pltpu.async_copy(utab_hbm.at[pl.ds(uid, 1)],
                             du.at[pl.ds(p, 1)], su)
            pltpu.async_copy(mtab_hbm.at[pl.ds(mid, 1)],
                             dm.at[pl.ds(p, 1)], sm)
            return 0

        lax.fori_loop(0, BLK, fire, 0)

    def drain_block(du, dm, su, sm):
        pltpu.make_async_copy(utab_hbm.at[pl.ds(0, BLK)], du, su).wait()
        pltpu.make_async_copy(mtab_hbm.at[pl.ds(0, BLK)], dm, sm).wait()

    def compute_block(b0, du, dm):
        def group(g, _):
            for i in range(L):
                pv = jnp.full((L,), g * L + i, jnp.int32)
                ua = plsc.load_gather(du, [pv, c0])
                ub = plsc.load_gather(du, [pv, c1])
                uc = plsc.load_gather(du, [pv, c2])
                ud = plsc.load_gather(du, [pv, c3])
                ma = plsc.load_gather(dm, [pv, c0])
                mb = plsc.load_gather(dm, [pv, c1])
                mc = plsc.load_gather(dm, [pv, c2])
                md = plsc.load_gather(dm, [pv, c3])
                acc = ((ua * ma) * wa + (ub * mb) * wb
                       + (uc * mc) * wc + (ud * md) * wd)
                mat_v[pl.ds(i * L, L)] = acc
            cols = [plsc.load_gather(mat_v, [base16 + l]) for l in range(L)]
            while len(cols) > 1:
                cols = [a + b for a, b in zip(cols[0::2], cols[1::2])]
            s = cols[0]
            out_v[pl.ds(b0 + g * L, L)] = 1.0 / (1.0 + jnp.exp(-(s + bias)))
            return 0

        lax.fori_loop(0, BLK // L, group, 0)

    bufs = [(dat_ua, dat_ma, sem_u, sem_m), (dat_ub, dat_mb, sem_u2, sem_m2)]
    n_blk = b_per_w // BLK
    fire_block(0, *bufs[0])
    for blk in range(n_blk):
        cur = bufs[blk % 2]
        if blk + 1 < n_blk:
            fire_block((blk + 1) * BLK, *bufs[(blk + 1) % 2])
        drain_block(*cur)
        compute_block(blk * BLK, cur[0], cur[1])

    pltpu.sync_copy(out_v, out_hbm.at[pl.ds(base, b_per_w)])


def kernel(train_data, user_embedding, movie_embedding, W_out, b_out):
    B = train_data.shape[0]
    b_per_w = B // NW
    w = W_out[:, 0]
    w4 = jnp.concatenate([
        w[0:L], w[L:2 * L],
        w[2 * L:2 * L + 2], jnp.zeros((14,), jnp.float32),
        w[D - L:D],
    ])
    flat_idx = train_data.reshape(-1).astype(jnp.int32)

    mesh = plsc.VectorSubcoreMesh(
        core_axis_name="c", subcore_axis_name="s",
        num_cores=NC, num_subcores=NS)

    run = functools.partial(
        pl.kernel,
        out_type=jax.ShapeDtypeStruct((B,), jnp.float32),
        mesh=mesh,
        compiler_params=pltpu.CompilerParams(
            needs_layout_passes=False, use_tc_tiling_on_sc=True),
        scratch_types=[
            pltpu.VMEM((2 * b_per_w,), jnp.int32),
            pltpu.VMEM((b_per_w,), jnp.int32),
            pltpu.VMEM((b_per_w,), jnp.int32),
            pltpu.VMEM((BLK, D), jnp.float32),
            pltpu.VMEM((BLK, D), jnp.float32),
            pltpu.VMEM((BLK, D), jnp.float32),
            pltpu.VMEM((BLK, D), jnp.float32),
            pltpu.VMEM((b_per_w,), jnp.float32),
            pltpu.VMEM((L * L,), jnp.float32),
            pltpu.VMEM((4 * L,), jnp.float32),
            pltpu.VMEM((L,), jnp.float32),
            pltpu.SemaphoreType.DMA,
            pltpu.SemaphoreType.DMA,
            pltpu.SemaphoreType.DMA,
            pltpu.SemaphoreType.DMA,
        ],
    )(functools.partial(_sc_body, b_per_w))

    out = run(flat_idx, user_embedding, movie_embedding, w4,
              jnp.broadcast_to(b_out.astype(jnp.float32), (L,)))
    return out.reshape(B, 1)
